# Initial kernel scaffold; baseline (speedup 1.0000x reference)
import jax
import jax.numpy as jnp
from jax.experimental import pallas as pl


def kernel(x, mask):
    raise NotImplementedError("write your pallas kernel here")


# speedup vs baseline: 2.5848x; 2.5848x over previous
import jax
import jax.numpy as jnp
from jax.experimental import pallas as pl
from jax.experimental.pallas import tpu as pltpu

_ROWS = 16384
_COLS = 4096
_BLOCK_ROWS = 512


def _zero_cols_kernel(mask_ref, x_ref, o_ref, keep_ref):
    @pl.when(pl.program_id(0) == 0)
    def _build_keep():
        cols = jax.lax.broadcasted_iota(jnp.int32, (_COLS, 1), 0)
        hit = (cols == mask_ref[...]).any(axis=1)
        keep_ref[...] = jnp.where(hit, 0.0, 1.0).reshape(1, _COLS)

    o_ref[...] = x_ref[...] * keep_ref[...]


def kernel(x, mask):
    mask2d = mask.reshape(1, -1)
    grid = (_ROWS // _BLOCK_ROWS,)
    return pl.pallas_call(
        _zero_cols_kernel,
        grid=grid,
        in_specs=[
            pl.BlockSpec((1, mask2d.shape[1]), lambda i: (0, 0)),
            pl.BlockSpec((_BLOCK_ROWS, _COLS), lambda i: (i, 0)),
        ],
        out_specs=pl.BlockSpec((_BLOCK_ROWS, _COLS), lambda i: (i, 0)),
        out_shape=jax.ShapeDtypeStruct((_ROWS, _COLS), x.dtype),
        scratch_shapes=[pltpu.VMEM((1, _COLS), jnp.float32)],
    )(mask2d, x)
